# baseline (speedup 1.0000x reference)
import functools

import jax
import jax.numpy as jnp
from jax import lax
from jax.experimental import pallas as pl
from jax.experimental.pallas import tpu as pltpu
from jax.experimental.pallas import tpu_sc as plsc

N = 10000
PN = 10240
E = 160000
D = 256
DH = 128
NC = 2
NS = 16
EPS = 10240
EP = EPS * NS
K = 128
CH = EPS // K
IB = 16
NB = CH // IB
K2 = 128
CH2 = EP // (NC * NS) // K2
GRP = 5
RPS = PN // NS


def _sc_agg_body(tbl, srcs, dsts, z128, s_out, src_v, dst_v, rows_v, acc,
                 sem0, sem1):
    c = lax.axis_index("c")
    s = lax.axis_index("s")

    pltpu.sync_copy(z128.at[pl.ds(s * RPS, RPS)], acc.at[pl.ds(s * RPS, RPS)])
    plsc.subcore_barrier()

    sems = (sem0, sem1)

    def _gather(i, b):
        pltpu.async_copy(tbl.at[src_v.at[i]], rows_v.at[b], sems[b])

    def _drain(i, b):
        pltpu.make_async_copy(tbl.at[src_v.at[i]], rows_v.at[b],
                              sems[b]).wait()

    def _scatter(i, b):
        pltpu.sync_copy(rows_v.at[b], acc.at[dst_v.at[i]], add=True)

    def _block(bi, carry):
        pltpu.sync_copy(srcs.at[c * NS + s, bi], src_v)
        pltpu.sync_copy(dsts.at[s, bi], dst_v)
        _gather(0, 0)

        def _pair(j, carry):
            i0 = 2 * j
            _gather(i0 + 1, 1)
            _drain(i0, 0)
            _scatter(i0, 0)

            @pl.when(i0 + 2 < IB)
            def _():
                _gather(i0 + 2, 0)
            _drain(i0 + 1, 1)
            _scatter(i0 + 1, 1)
            return carry
        lax.fori_loop(0, IB // 2, _pair, 0)
        return carry
    lax.fori_loop(0, NB, _block, 0)
    plsc.subcore_barrier()

    pltpu.sync_copy(acc.at[pl.ds(s * RPS, RPS)],
                    s_out.at[pl.ds(c * PN + s * RPS, RPS)])


def _sc_cnt_body(dsts, ones, z128, cnt_out, dst_v, ones_v, cnt_acc, semc):
    c = lax.axis_index("c")
    s = lax.axis_index("s")

    pltpu.sync_copy(z128.at[pl.ds(s * RPS, RPS)],
                    cnt_acc.at[pl.ds(s * RPS, RPS)])
    pltpu.sync_copy(dsts.at[c * NS + s], dst_v)
    pltpu.sync_copy(ones, ones_v)
    plsc.subcore_barrier()

    def _grp(g, carry):
        for b in range(GRP):
            pltpu.async_copy(ones_v, cnt_acc.at[dst_v.at[GRP * g + b]],
                             semc, add=True)
        for b in range(GRP):
            pltpu.make_async_copy(ones_v, cnt_acc.at[dst_v.at[GRP * g + b]],
                                  semc).wait()
        return carry
    lax.fori_loop(0, CH2 // GRP, _grp, 0)
    plsc.subcore_barrier()

    pltpu.sync_copy(cnt_acc.at[pl.ds(s * RPS, RPS)],
                    cnt_out.at[pl.ds(c * PN + s * RPS, RPS)])


@functools.cache
def _sc_kernels():
    mesh = plsc.VectorSubcoreMesh(core_axis_name="c", subcore_axis_name="s",
                                  num_cores=NC, num_subcores=NS)
    agg = pl.kernel(
        _sc_agg_body,
        out_type=[jax.ShapeDtypeStruct((NC * PN, DH), jnp.float32)],
        mesh=mesh,
        scratch_types=[
            pltpu.VMEM((IB, K), jnp.int32),
            pltpu.VMEM((IB, K), jnp.int32),
            pltpu.VMEM((2, K, DH), jnp.float32),
            pltpu.VMEM_SHARED((PN, DH), jnp.float32),
            pltpu.SemaphoreType.DMA,
            pltpu.SemaphoreType.DMA,
        ],
    )
    cnt = pl.kernel(
        _sc_cnt_body,
        out_type=[jax.ShapeDtypeStruct((NC * PN, DH), jnp.float32)],
        mesh=mesh,
        scratch_types=[
            pltpu.VMEM((CH2, K2), jnp.int32),
            pltpu.VMEM((K2, DH), jnp.float32),
            pltpu.VMEM_SHARED((PN, DH), jnp.float32),
            pltpu.SemaphoreType.DMA,
        ],
    )
    return agg, cnt


def _tc1_body(s_ref, cnt_ref, x_ref, wl_ref, bl_ref, wr_ref, h_ref):
    agg = jnp.concatenate([s_ref[0], s_ref[1]], axis=1)
    deg = cnt_ref[0, :, 0:1] + cnt_ref[1, :, 0:1]
    inv = 1.0 / jnp.maximum(deg, 1.0)
    h = jnp.dot(agg, wl_ref[...], preferred_element_type=jnp.float32) * inv
    h = h + bl_ref[...] + jnp.dot(x_ref[...], wr_ref[...],
                                  preferred_element_type=jnp.float32)
    h = jnp.maximum(h, 0.0)
    h_ref[0] = h[:, :DH]
    h_ref[1] = h[:, DH:]


def _tc2_body(s_ref, cnt_ref, h_ref, wl_ref, bl_ref, wr_ref, wo_ref, bo_ref,
              o_ref):
    agg = jnp.concatenate([s_ref[0], s_ref[1]], axis=1)
    hin = jnp.concatenate([h_ref[0], h_ref[1]], axis=1)
    deg = cnt_ref[0, :, 0:1] + cnt_ref[1, :, 0:1]
    inv = 1.0 / jnp.maximum(deg, 1.0)
    t = jnp.dot(agg, wl_ref[...], preferred_element_type=jnp.float32) * inv
    t = t + bl_ref[...] + jnp.dot(hin, wr_ref[...],
                                  preferred_element_type=jnp.float32)
    t = jnp.maximum(t, 0.0)
    o_ref[...] = jnp.dot(t, wo_ref[...],
                         preferred_element_type=jnp.float32) + bo_ref[...]


_B = 1000
_G = N // _B

_spec_S = pl.BlockSpec((NC, _B, DH), lambda i: (0, i, 0))
_spec_cnt = pl.BlockSpec((NC, _B, DH), lambda i: (0, i, 0))
_spec_rows = pl.BlockSpec((_B, D), lambda i: (i, 0))
_spec_W = pl.BlockSpec((D, D), lambda i: (0, 0))
_spec_b = pl.BlockSpec((1, D), lambda i: (0, 0))

_tc1 = pl.pallas_call(
    _tc1_body,
    grid=(_G,),
    in_specs=[_spec_S, _spec_cnt, _spec_rows, _spec_W, _spec_b, _spec_W],
    out_specs=_spec_S,
    out_shape=jax.ShapeDtypeStruct((NC, PN, DH), jnp.float32),
)

_tc2 = pl.pallas_call(
    _tc2_body,
    grid=(_G,),
    in_specs=[_spec_S, _spec_cnt, _spec_S, _spec_W, _spec_b, _spec_W,
              _spec_W, _spec_b],
    out_specs=_spec_rows,
    out_shape=jax.ShapeDtypeStruct((N, D), jnp.float32),
)


def kernel(x, edge_index, Wl1, bl1, Wr1, Wl2, bl2, Wr2, Wo, bo):
    src = edge_index[0].astype(jnp.int32)
    dst = edge_index[1].astype(jnp.int32)
    pad = EP - E
    srcp = jnp.concatenate([src, jnp.zeros((pad,), jnp.int32)])
    dstp = jnp.concatenate([dst, jnp.full((pad,), PN - 1, jnp.int32)])
    srcs = jnp.stack([srcp, srcp + PN]).reshape(NC * NS, NB, IB, K)
    dsts = dstp.reshape(NS, NB, IB, K)
    dsts2 = dstp.reshape(NC * NS, CH2, K2)
    z128 = jnp.zeros((PN, DH), jnp.float32)
    ones = jnp.ones((K2, DH), jnp.float32)

    zrow = jnp.zeros((PN - N, DH), jnp.float32)
    xs = jnp.concatenate([x[:, :DH], zrow, x[:, DH:], zrow])
    agg_fn, cnt_fn = _sc_kernels()
    (cnt,) = cnt_fn(dsts2, ones, z128)
    z128dep = z128 + cnt[0, 0] * 0.0
    (s1,) = agg_fn(xs, srcs, dsts, z128dep)
    hs = _tc1(s1.reshape(NC, PN, DH), cnt.reshape(NC, PN, DH), x, Wl1,
              bl1.reshape(1, D), Wr1)
    (s2,) = agg_fn(hs.reshape(NC * PN, DH), srcs, dsts, z128)
    out = _tc2(s2.reshape(NC, PN, DH), cnt.reshape(NC, PN, DH), hs, Wl2,
               bl2.reshape(1, D), Wr2, Wo, bo.reshape(1, D))
    return out


# speedup vs baseline: 2.1138x; 2.1138x over previous
import functools

import jax
import jax.numpy as jnp
from jax import lax
from jax.experimental import pallas as pl
from jax.experimental.pallas import tpu as pltpu
from jax.experimental.pallas import tpu_sc as plsc

N = 10000
PN = 10240
E = 160000
D = 256
DH = 128
NC = 2
NS = 16
K = 80
CH = E // NS // K
IB = 25
NB = CH // IB
K2 = 40
CH2 = E // (NC * NS) // K2
GRP = 5
RPS = PN // NS


def _sc_agg_body(tbl, srcs, dsts, z128, s_out, src_v, dst_v, rows_v, acc,
                 sem0, sem1):
    c = lax.axis_index("c")
    s = lax.axis_index("s")

    pltpu.sync_copy(z128.at[pl.ds(s * RPS, RPS)], acc.at[pl.ds(s * RPS, RPS)])
    plsc.subcore_barrier()

    sems = (sem0, sem1)

    def _gather(i, b):
        pltpu.async_copy(tbl.at[src_v.at[i]], rows_v.at[b], sems[b])

    def _drain(i, b):
        pltpu.make_async_copy(tbl.at[src_v.at[i]], rows_v.at[b],
                              sems[b]).wait()

    def _scatter(i, b):
        pltpu.sync_copy(rows_v.at[b], acc.at[dst_v.at[i]], add=True)

    def _block(bi, carry):
        pltpu.sync_copy(srcs.at[c * NS + s, bi], src_v)
        pltpu.sync_copy(dsts.at[s, bi], dst_v)
        _gather(0, 0)

        def _pair(j, carry):
            i0 = 2 * j
            _gather(i0 + 1, 1)
            _drain(i0, 0)
            _scatter(i0, 0)

            @pl.when(i0 + 2 < IB)
            def _():
                _gather(i0 + 2, 0)
            _drain(i0 + 1, 1)
            _scatter(i0 + 1, 1)
            return carry
        lax.fori_loop(0, (IB - 1) // 2, _pair, 0)
        _drain(IB - 1, 0)
        _scatter(IB - 1, 0)
        return carry
    lax.fori_loop(0, NB, _block, 0)
    plsc.subcore_barrier()

    pltpu.sync_copy(acc.at[pl.ds(s * RPS, RPS)],
                    s_out.at[pl.ds(c * PN + s * RPS, RPS)])


def _sc_cnt_body(dsts, ones, z128, cnt_out, dst_v, ones_v, cnt_acc, semc):
    c = lax.axis_index("c")
    s = lax.axis_index("s")

    pltpu.sync_copy(z128.at[pl.ds(s * RPS, RPS)],
                    cnt_acc.at[pl.ds(s * RPS, RPS)])
    pltpu.sync_copy(dsts.at[c * NS + s], dst_v)
    pltpu.sync_copy(ones, ones_v)
    plsc.subcore_barrier()

    def _grp(g, carry):
        for b in range(GRP):
            pltpu.async_copy(ones_v, cnt_acc.at[dst_v.at[GRP * g + b]],
                             semc, add=True)
        for b in range(GRP):
            pltpu.make_async_copy(ones_v, cnt_acc.at[dst_v.at[GRP * g + b]],
                                  semc).wait()
        return carry
    lax.fori_loop(0, CH2 // GRP, _grp, 0)
    plsc.subcore_barrier()

    pltpu.sync_copy(cnt_acc.at[pl.ds(s * RPS, RPS)],
                    cnt_out.at[pl.ds(c * PN + s * RPS, RPS)])


@functools.cache
def _sc_kernels():
    mesh = plsc.VectorSubcoreMesh(core_axis_name="c", subcore_axis_name="s",
                                  num_cores=NC, num_subcores=NS)
    agg = pl.kernel(
        _sc_agg_body,
        out_type=[jax.ShapeDtypeStruct((NC * PN, DH), jnp.float32)],
        mesh=mesh,
        scratch_types=[
            pltpu.VMEM((IB, K), jnp.int32),
            pltpu.VMEM((IB, K), jnp.int32),
            pltpu.VMEM((2, K, DH), jnp.float32),
            pltpu.VMEM_SHARED((PN, DH), jnp.float32),
            pltpu.SemaphoreType.DMA,
            pltpu.SemaphoreType.DMA,
        ],
    )
    cnt = pl.kernel(
        _sc_cnt_body,
        out_type=[jax.ShapeDtypeStruct((NC * PN, DH), jnp.float32)],
        mesh=mesh,
        scratch_types=[
            pltpu.VMEM((CH2, K2), jnp.int32),
            pltpu.VMEM((K2, DH), jnp.float32),
            pltpu.VMEM_SHARED((PN, DH), jnp.float32),
            pltpu.SemaphoreType.DMA,
        ],
    )
    return agg, cnt


def _tc1_body(s_ref, cnt_ref, x_ref, wl_ref, bl_ref, wr_ref, h_ref):
    agg = jnp.concatenate([s_ref[0], s_ref[1]], axis=1)
    deg = cnt_ref[0, :, 0:1] + cnt_ref[1, :, 0:1]
    inv = 1.0 / jnp.maximum(deg, 1.0)
    h = jnp.dot(agg, wl_ref[...], preferred_element_type=jnp.float32) * inv
    h = h + bl_ref[...] + jnp.dot(x_ref[...], wr_ref[...],
                                  preferred_element_type=jnp.float32)
    h = jnp.maximum(h, 0.0)
    h_ref[0] = h[:, :DH]
    h_ref[1] = h[:, DH:]


def _tc2_body(s_ref, cnt_ref, h_ref, wl_ref, bl_ref, wr_ref, wo_ref, bo_ref,
              o_ref):
    agg = jnp.concatenate([s_ref[0], s_ref[1]], axis=1)
    hin = jnp.concatenate([h_ref[0], h_ref[1]], axis=1)
    deg = cnt_ref[0, :, 0:1] + cnt_ref[1, :, 0:1]
    inv = 1.0 / jnp.maximum(deg, 1.0)
    t = jnp.dot(agg, wl_ref[...], preferred_element_type=jnp.float32) * inv
    t = t + bl_ref[...] + jnp.dot(hin, wr_ref[...],
                                  preferred_element_type=jnp.float32)
    t = jnp.maximum(t, 0.0)
    o_ref[...] = jnp.dot(t, wo_ref[...],
                         preferred_element_type=jnp.float32) + bo_ref[...]


_B = 1000
_G = N // _B

_spec_S = pl.BlockSpec((NC, _B, DH), lambda i: (0, i, 0))
_spec_cnt = pl.BlockSpec((NC, _B, DH), lambda i: (0, i, 0))
_spec_rows = pl.BlockSpec((_B, D), lambda i: (i, 0))
_spec_W = pl.BlockSpec((D, D), lambda i: (0, 0))
_spec_b = pl.BlockSpec((1, D), lambda i: (0, 0))

_tc1 = pl.pallas_call(
    _tc1_body,
    grid=(_G,),
    in_specs=[_spec_S, _spec_cnt, _spec_rows, _spec_W, _spec_b, _spec_W],
    out_specs=_spec_S,
    out_shape=jax.ShapeDtypeStruct((NC, PN, DH), jnp.float32),
)

_tc2 = pl.pallas_call(
    _tc2_body,
    grid=(_G,),
    in_specs=[_spec_S, _spec_cnt, _spec_S, _spec_W, _spec_b, _spec_W,
              _spec_W, _spec_b],
    out_specs=_spec_rows,
    out_shape=jax.ShapeDtypeStruct((N, D), jnp.float32),
)


def kernel(x, edge_index, Wl1, bl1, Wr1, Wl2, bl2, Wr2, Wo, bo):
    src = edge_index[0].astype(jnp.int32)
    dst = edge_index[1].astype(jnp.int32)
    srcs = jnp.stack([src, src + PN]).reshape(NC * NS, NB, IB, K)
    dsts = dst.reshape(NS, NB, IB, K)
    dsts2 = dst.reshape(NC * NS, CH2, K2)
    z128 = jnp.zeros((PN, DH), jnp.float32)
    ones = jnp.ones((K2, DH), jnp.float32)

    zrow = jnp.zeros((PN - N, DH), jnp.float32)
    xs = jnp.concatenate([x[:, :DH], zrow, x[:, DH:], zrow])
    agg_fn, cnt_fn = _sc_kernels()
    (cnt,) = cnt_fn(dsts2, ones, z128)
    z128dep = z128 + cnt[0, 0] * 0.0
    (s1,) = agg_fn(xs, srcs, dsts, z128dep)
    hs = _tc1(s1.reshape(NC, PN, DH), cnt.reshape(NC, PN, DH), x, Wl1,
              bl1.reshape(1, D), Wr1)
    (s2,) = agg_fn(hs.reshape(NC * PN, DH), srcs, dsts, z128)
    out = _tc2(s2.reshape(NC, PN, DH), cnt.reshape(NC, PN, DH), hs, Wl2,
               bl2.reshape(1, D), Wr2, Wo, bo.reshape(1, D))
    return out
